# baseline (device time: 6585 ns/iter reference)
import jax
import jax.numpy as jnp
from jax import lax
from jax.experimental import pallas as pl
from jax.experimental.pallas import tpu as pltpu

N_CHUNKS = 2


def kernel(x):
    m, n = x.shape
    ch = m // N_CHUNKS

    def body(
        x_ref, out_ref,
        qx_ref, sc_ref, comm_q_ref, comm_s_ref,
        q_send_sems, q_recv_sems, s_send_sems, s_recv_sems,
    ):
        my_x = lax.axis_index("x")
        my_y = lax.axis_index("y")
        my_z = lax.axis_index("z")
        partner = (1 - my_x, my_y, my_z)

        barrier_sem = pltpu.get_barrier_semaphore()
        pl.semaphore_signal(
            barrier_sem, inc=1,
            device_id=partner, device_id_type=pl.DeviceIdType.MESH,
        )

        def quantize(c):
            chunk = x_ref[pl.ds(c * ch, ch), :]
            scale = jnp.max(jnp.abs(chunk)) / 127.0 + 1e-30
            sc_ref[c, 0] = scale
            qx_ref[c, :, :] = jnp.round(chunk / scale).astype(jnp.int8)

        def start_send(c):
            rs = pltpu.make_async_remote_copy(
                src_ref=sc_ref.at[c],
                dst_ref=comm_s_ref.at[c],
                send_sem=s_send_sems.at[c],
                recv_sem=s_recv_sems.at[c],
                device_id=partner,
                device_id_type=pl.DeviceIdType.MESH,
            )
            rs.start()
            rq = pltpu.make_async_remote_copy(
                src_ref=qx_ref.at[c],
                dst_ref=comm_q_ref.at[c],
                send_sem=q_send_sems.at[c],
                recv_sem=q_recv_sems.at[c],
                device_id=partner,
                device_id_type=pl.DeviceIdType.MESH,
            )
            rq.start()
            return rs, rq

        quantize(0)
        pl.semaphore_wait(barrier_sem, 1)
        rdmas = [start_send(0)]
        for c in range(1, N_CHUNKS):
            quantize(c)
            rdmas.append(start_send(c))

        for c in range(N_CHUNKS):
            rs, rq = rdmas[c]
            rs.wait_recv()
            rq.wait_recv()
            out_ref[pl.ds(c * ch, ch), :] = (
                x_ref[pl.ds(c * ch, ch), :]
                + comm_q_ref[c, :, :].astype(jnp.float32) * comm_s_ref[c, 0]
            )
        for rs, rq in rdmas:
            rs.wait_send()
            rq.wait_send()

    return pl.pallas_call(
        body,
        out_shape=jax.ShapeDtypeStruct((m, n), x.dtype),
        in_specs=[pl.BlockSpec(memory_space=pltpu.VMEM)],
        out_specs=pl.BlockSpec(memory_space=pltpu.VMEM),
        scratch_shapes=[
            pltpu.VMEM((N_CHUNKS, ch, n), jnp.int8),
            pltpu.SMEM((N_CHUNKS, 128), jnp.float32),
            pltpu.VMEM((N_CHUNKS, ch, n), jnp.int8),
            pltpu.SMEM((N_CHUNKS, 128), jnp.float32),
            pltpu.SemaphoreType.DMA((N_CHUNKS,)),
            pltpu.SemaphoreType.DMA((N_CHUNKS,)),
            pltpu.SemaphoreType.DMA((N_CHUNKS,)),
            pltpu.SemaphoreType.DMA((N_CHUNKS,)),
        ],
        compiler_params=pltpu.CompilerParams(collective_id=0),
    )(x)
